# baseline (speedup 1.0000x reference)
import functools

import jax
import jax.numpy as jnp
from jax import lax
from jax.experimental import pallas as pl
from jax.experimental.pallas import tpu as pltpu

N = 10000
E = 320000
D_IN = 128
D_EDGE = 16
HID = 128
D_OUT = 128

MLP_BLOCK = 512


def _xp_body(x_ref, w_ref, b_ref, o_ref):
    o_ref[...] = (
        jnp.dot(x_ref[...], w_ref[...], preferred_element_type=jnp.float32)
        + b_ref[...]
    )


def _node_precompute(x, W1a, b1):
    grid = (10,)
    return pl.pallas_call(
        _xp_body,
        grid=grid,
        in_specs=[
            pl.BlockSpec((N // 10, D_IN), lambda i: (i, 0)),
            pl.BlockSpec((D_IN, HID), lambda i: (0, 0)),
            pl.BlockSpec((1, HID), lambda i: (0, 0)),
        ],
        out_specs=pl.BlockSpec((N // 10, HID), lambda i: (i, 0)),
        out_shape=jax.ShapeDtypeStruct((N, HID), jnp.float32),
    )(x, W1a, b1)


def _mlp_body(e1_ref, ea_ref, w1b_ref, w2_ref, b2_ref, w3_ref, b3_ref,
              w4_ref, b4_ref, m_ref):
    h1 = jnp.maximum(
        e1_ref[...]
        + jnp.dot(ea_ref[...], w1b_ref[...], preferred_element_type=jnp.float32),
        0.0,
    )
    h2 = jnp.maximum(
        jnp.dot(h1, w2_ref[...], preferred_element_type=jnp.float32) + b2_ref[...],
        0.0,
    )
    h3 = jnp.maximum(
        jnp.dot(h2, w3_ref[...], preferred_element_type=jnp.float32) + b3_ref[...],
        0.0,
    )
    m_ref[...] = jnp.maximum(
        jnp.dot(h3, w4_ref[...], preferred_element_type=jnp.float32) + b4_ref[...],
        0.0,
    )


def _edge_mlp(e1, edge_attr, W1b, W2, b2, W3, b3, W4, b4):
    grid = (E // MLP_BLOCK,)
    return pl.pallas_call(
        _mlp_body,
        grid=grid,
        in_specs=[
            pl.BlockSpec((MLP_BLOCK, HID), lambda i: (i, 0)),
            pl.BlockSpec((MLP_BLOCK, D_EDGE), lambda i: (i, 0)),
            pl.BlockSpec((D_EDGE, HID), lambda i: (0, 0)),
            pl.BlockSpec((HID, 2 * HID), lambda i: (0, 0)),
            pl.BlockSpec((1, 2 * HID), lambda i: (0, 0)),
            pl.BlockSpec((2 * HID, HID), lambda i: (0, 0)),
            pl.BlockSpec((1, HID), lambda i: (0, 0)),
            pl.BlockSpec((HID, D_OUT), lambda i: (0, 0)),
            pl.BlockSpec((1, D_OUT), lambda i: (0, 0)),
        ],
        out_specs=pl.BlockSpec((MLP_BLOCK, D_OUT), lambda i: (i, 0)),
        out_shape=jax.ShapeDtypeStruct((E, D_OUT), jnp.float32),
    )(e1, edge_attr, W1b, W2, b2, W3, b3, W4, b4)


def _combine_body(s_ref, c_ref, o_ref):
    total = s_ref[0] + s_ref[1]
    cnt = jnp.sum(c_ref[...], axis=0)
    denom = jnp.maximum(cnt, 1.0)
    o_ref[...] = total / denom[:, None]


def _combine(sums, cnt):
    NP = 10240
    sums_p = jnp.pad(sums, ((0, 0), (0, NP - N), (0, 0)))
    cnt_p = jnp.pad(cnt, ((0, 0), (0, NP - N)))
    grid = (10,)
    blk = NP // 10
    out = pl.pallas_call(
        _combine_body,
        grid=grid,
        in_specs=[
            pl.BlockSpec((2, blk, D_OUT), lambda i: (0, i, 0)),
            pl.BlockSpec((32, blk), lambda i: (0, i)),
        ],
        out_specs=pl.BlockSpec((blk, D_OUT), lambda i: (i, 0)),
        out_shape=jax.ShapeDtypeStruct((NP, D_OUT), jnp.float32),
    )(sums_p, cnt_p)
    return out[:N]


def kernel(x, edge_index, edge_attr, W1, b1, W2, b2, W3, b3, W4, b4):
    dst = edge_index[1].astype(jnp.int32)
    W1a = W1[:D_IN]
    W1b = W1[D_IN:]
    b1r = b1.reshape(1, HID)
    b2r = b2.reshape(1, 2 * HID)
    b3r = b3.reshape(1, HID)
    b4r = b4.reshape(1, D_OUT)

    xp = _node_precompute(x, W1a, b1r)

    e1 = jnp.take(xp, dst, axis=0)
    m = _edge_mlp(e1, edge_attr, W1b, W2, b2r, W3, b3r, W4, b4r)
    sums = jax.ops.segment_sum(m, dst, num_segments=N)
    cnt = jax.ops.segment_sum(jnp.ones((E,), jnp.float32), dst, num_segments=N)
    sums2 = jnp.stack([sums, jnp.zeros_like(sums)])
    cnt32 = jnp.zeros((32, N), jnp.float32).at[0].set(cnt)
    return _combine(sums2, cnt32)


# speedup vs baseline: 2.3749x; 2.3749x over previous
import functools

import jax
import jax.numpy as jnp
from jax import lax
from jax.experimental import pallas as pl
from jax.experimental.pallas import tpu as pltpu
from jax.experimental.pallas import tpu_sc as plsc

N = 10000
E = 320000
D_IN = 128
D_EDGE = 16
HID = 128
D_OUT = 128

MLP_BLOCK = 512

NC = 2
NS = 16
NW = NC * NS
EPW = E // NW
KG = 80
NITER = EPW // KG
NP = 10240
CH = 128
HR = 1280

_SC_MESH = plsc.VectorSubcoreMesh(
    core_axis_name="c", subcore_axis_name="s", num_cores=NC, num_subcores=NS
)
_SC_PARAMS = pltpu.CompilerParams(needs_layout_passes=False)


def _sc_gather_body(xp_hbm, dst_hbm, e1_hbm, cnt_hbm, idx_v, rows_v, hist_v, sem):
    c = lax.axis_index("c")
    s = lax.axis_index("s")
    wid = c * NS + s
    base = wid * EPW

    zeros16 = jnp.zeros((16,), jnp.float32)

    def zero_hist(i, carry):
        hist_v[0, pl.ds(i * 16, 16)] = zeros16
        hist_v[1, pl.ds(i * 16, 16)] = zeros16
        hist_v[2, pl.ds(i * 16, 16)] = zeros16
        hist_v[3, pl.ds(i * 16, 16)] = zeros16
        hist_v[4, pl.ds(i * 16, 16)] = zeros16
        hist_v[5, pl.ds(i * 16, 16)] = zeros16
        hist_v[6, pl.ds(i * 16, 16)] = zeros16
        hist_v[7, pl.ds(i * 16, 16)] = zeros16
        return carry

    lax.fori_loop(0, HR // 16, zero_hist, 0)

    ones = jnp.ones((16,), jnp.float32)

    def body(i, carry):
        off = base + i * KG
        pltpu.sync_copy(dst_hbm.at[pl.ds(off, KG)], idx_v)
        pltpu.async_copy(xp_hbm.at[idx_v], rows_v, sem).wait()
        pltpu.sync_copy(rows_v, e1_hbm.at[pl.ds(off, KG)])
        for j in range(KG // 16):
            idxs = idx_v[pl.ds(j * 16, 16)]
            plsc.addupdate_scatter(hist_v, [idxs // HR, idxs % HR], ones)
        return carry

    lax.fori_loop(0, NITER, body, 0)
    pltpu.sync_copy(hist_v, cnt_hbm.at[wid])


@functools.partial(
    pl.kernel,
    out_type=(
        jax.ShapeDtypeStruct((E, HID), jnp.float32),
        jax.ShapeDtypeStruct((NW, 8, HR), jnp.float32),
    ),
    mesh=_SC_MESH,
    scratch_types=[
        pltpu.VMEM((KG,), jnp.int32),
        pltpu.VMEM((KG, HID), jnp.float32),
        pltpu.VMEM((8, HR), jnp.float32),
        pltpu.SemaphoreType.DMA,
    ],
    compiler_params=_SC_PARAMS,
)
def _sc_gather(*refs):
    _sc_gather_body(*refs)


def _sc_scatter_body(m_hbm, dst_hbm, z_hbm, sums_hbm, idx_v, rows_v, buf_v, acc_sh):
    c = lax.axis_index("c")
    s = lax.axis_index("s")
    wid = c * NS + s
    base = wid * EPW

    pltpu.sync_copy(z_hbm, buf_v)
    for k in range(5):
        r0 = s * (5 * CH) + k * CH
        pltpu.sync_copy(buf_v, acc_sh.at[pl.ds(r0, CH)])
    plsc.subcore_barrier()

    def body(i, carry):
        off = base + i * KG
        pltpu.sync_copy(dst_hbm.at[pl.ds(off, KG)], idx_v)
        pltpu.sync_copy(m_hbm.at[pl.ds(off, KG)], rows_v)
        pltpu.sync_copy(rows_v, acc_sh.at[idx_v], add=True)
        return carry

    lax.fori_loop(0, NITER, body, 0)
    plsc.subcore_barrier()

    for k in range(5):
        r0 = s * (5 * CH) + k * CH
        pltpu.sync_copy(acc_sh.at[pl.ds(r0, CH)], buf_v)
        pltpu.sync_copy(buf_v, sums_hbm.at[c, pl.ds(r0, CH)])


@functools.partial(
    pl.kernel,
    out_type=jax.ShapeDtypeStruct((NC, NP, HID), jnp.float32),
    mesh=_SC_MESH,
    scratch_types=[
        pltpu.VMEM((KG,), jnp.int32),
        pltpu.VMEM((KG, HID), jnp.float32),
        pltpu.VMEM((CH, HID), jnp.float32),
        pltpu.VMEM_SHARED((NP, HID), jnp.float32),
    ],
    compiler_params=_SC_PARAMS,
)
def _sc_scatter(*refs):
    _sc_scatter_body(*refs)


def _xp_body(x_ref, w_ref, b_ref, o_ref):
    o_ref[...] = (
        jnp.dot(x_ref[...], w_ref[...], preferred_element_type=jnp.float32)
        + b_ref[...]
    )


def _node_precompute(x, W1a, b1):
    grid = (10,)
    return pl.pallas_call(
        _xp_body,
        grid=grid,
        in_specs=[
            pl.BlockSpec((N // 10, D_IN), lambda i: (i, 0)),
            pl.BlockSpec((D_IN, HID), lambda i: (0, 0)),
            pl.BlockSpec((1, HID), lambda i: (0, 0)),
        ],
        out_specs=pl.BlockSpec((N // 10, HID), lambda i: (i, 0)),
        out_shape=jax.ShapeDtypeStruct((N, HID), jnp.float32),
    )(x, W1a, b1)


def _mlp_body(e1_ref, ea_ref, w1b_ref, w2_ref, b2_ref, w3_ref, b3_ref,
              w4_ref, b4_ref, m_ref):
    h1 = jnp.maximum(
        e1_ref[...]
        + jnp.dot(ea_ref[...], w1b_ref[...], preferred_element_type=jnp.float32),
        0.0,
    )
    h2 = jnp.maximum(
        jnp.dot(h1, w2_ref[...], preferred_element_type=jnp.float32) + b2_ref[...],
        0.0,
    )
    h3 = jnp.maximum(
        jnp.dot(h2, w3_ref[...], preferred_element_type=jnp.float32) + b3_ref[...],
        0.0,
    )
    m_ref[...] = jnp.maximum(
        jnp.dot(h3, w4_ref[...], preferred_element_type=jnp.float32) + b4_ref[...],
        0.0,
    )


def _edge_mlp(e1, edge_attr, W1b, W2, b2, W3, b3, W4, b4):
    grid = (E // MLP_BLOCK,)
    return pl.pallas_call(
        _mlp_body,
        grid=grid,
        in_specs=[
            pl.BlockSpec((MLP_BLOCK, HID), lambda i: (i, 0)),
            pl.BlockSpec((MLP_BLOCK, D_EDGE), lambda i: (i, 0)),
            pl.BlockSpec((D_EDGE, HID), lambda i: (0, 0)),
            pl.BlockSpec((HID, 2 * HID), lambda i: (0, 0)),
            pl.BlockSpec((1, 2 * HID), lambda i: (0, 0)),
            pl.BlockSpec((2 * HID, HID), lambda i: (0, 0)),
            pl.BlockSpec((1, HID), lambda i: (0, 0)),
            pl.BlockSpec((HID, D_OUT), lambda i: (0, 0)),
            pl.BlockSpec((1, D_OUT), lambda i: (0, 0)),
        ],
        out_specs=pl.BlockSpec((MLP_BLOCK, D_OUT), lambda i: (i, 0)),
        out_shape=jax.ShapeDtypeStruct((E, D_OUT), jnp.float32),
    )(e1, edge_attr, W1b, W2, b2, W3, b3, W4, b4)


def _combine_body(s_ref, c_ref, o_ref):
    total = s_ref[0] + s_ref[1]
    cnt = jnp.sum(c_ref[...], axis=0)
    denom = jnp.maximum(cnt, 1.0)
    o_ref[...] = total / denom[:, None]


def _combine(sums, cnt):
    grid = (10,)
    blk = NP // 10
    out = pl.pallas_call(
        _combine_body,
        grid=grid,
        in_specs=[
            pl.BlockSpec((2, blk, D_OUT), lambda i: (0, i, 0)),
            pl.BlockSpec((32, blk), lambda i: (0, i)),
        ],
        out_specs=pl.BlockSpec((blk, D_OUT), lambda i: (i, 0)),
        out_shape=jax.ShapeDtypeStruct((NP, D_OUT), jnp.float32),
    )(sums, cnt)
    return out[:N]


def kernel(x, edge_index, edge_attr, W1, b1, W2, b2, W3, b3, W4, b4):
    dst = edge_index[1].astype(jnp.int32)
    W1a = W1[:D_IN]
    W1b = W1[D_IN:]
    b1r = b1.reshape(1, HID)
    b2r = b2.reshape(1, 2 * HID)
    b3r = b3.reshape(1, HID)
    b4r = b4.reshape(1, D_OUT)

    xp = _node_precompute(x, W1a, b1r)

    e1, cnt3 = _sc_gather(xp, dst)
    m = _edge_mlp(e1, edge_attr, W1b, W2, b2r, W3, b3r, W4, b4r)
    zeros = jnp.zeros((CH, HID), jnp.float32)
    sums = _sc_scatter(m, dst, zeros)
    cnt = cnt3.reshape(NW, NP)
    return _combine(sums, cnt)


# speedup vs baseline: 3.0745x; 1.2946x over previous
import functools

import jax
import jax.numpy as jnp
from jax import lax
from jax.experimental import pallas as pl
from jax.experimental.pallas import tpu as pltpu
from jax.experimental.pallas import tpu_sc as plsc

N = 10000
E = 320000
D_IN = 128
D_EDGE = 16
HID = 128
D_OUT = 128

MLP_BLOCK = 512

NC = 2
NS = 16
NW = NC * NS
EPW = E // NW
KG = 80
NITER = EPW // KG
NB = 5
NGRP = NITER // NB
NP = 10240
KS = 40
NITER_S = EPW // KS
NGRP_S = NITER_S // NB
RPT = NP // NS
HROWS = 5
HR = 2048

_SC_MESH = plsc.VectorSubcoreMesh(
    core_axis_name="c", subcore_axis_name="s", num_cores=NC, num_subcores=NS
)
_SC_PARAMS = pltpu.CompilerParams(needs_layout_passes=False)


def _sc_gather_body(xp_hbm, dst_hbm, e1_hbm, cnt_hbm, idx_v, rows, hist_v,
                    sem_i, sems_g, sems_w):
    c = lax.axis_index("c")
    s = lax.axis_index("s")
    wid = c * NS + s
    base = wid * EPW

    idx_load = pltpu.async_copy(dst_hbm.at[wid], idx_v, sem_i)

    zeros16 = jnp.zeros((16,), jnp.float32)

    def zero_hist(i, carry):
        for r in range(HROWS):
            hist_v[r, pl.ds(i * 16, 16)] = zeros16
        return carry

    lax.fori_loop(0, HR // 16, zero_hist, 0)
    idx_load.wait()

    ones = jnp.ones((16,), jnp.float32)

    def hist_update(i):
        for j in range(KG // 16):
            idxs = idx_v[i, pl.ds(j * 16, 16)]
            plsc.addupdate_scatter(
                hist_v, [lax.shift_right_logical(idxs, 11),
                         lax.bitwise_and(idxs, HR - 1)], ones)

    def group(g, carry):
        descs = []
        for b in range(NB):
            i = g * NB + b

            @pl.when(g > 0)
            def _():
                pltpu.make_async_copy(
                    rows[b], e1_hbm.at[pl.ds(base + i * KG, KG)], sems_w[b]
                ).wait()

            descs.append(
                pltpu.async_copy(xp_hbm.at[idx_v.at[i]], rows[b], sems_g[b])
            )
        for b in range(NB):
            i = g * NB + b
            hist_update(i)
            descs[b].wait()
            pltpu.async_copy(rows[b], e1_hbm.at[pl.ds(base + i * KG, KG)],
                             sems_w[b])
        return carry

    lax.fori_loop(0, NGRP, group, 0)

    for b in range(NB):
        pltpu.make_async_copy(
            rows[b], e1_hbm.at[pl.ds(base, KG)], sems_w[b]
        ).wait()
    pltpu.sync_copy(hist_v, cnt_hbm.at[wid])


@functools.partial(
    pl.kernel,
    out_type=(
        jax.ShapeDtypeStruct((E, HID), jnp.float32),
        jax.ShapeDtypeStruct((NW, HROWS, HR), jnp.float32),
    ),
    mesh=_SC_MESH,
    scratch_types=[
        pltpu.VMEM((NITER, KG), jnp.int32),
        [pltpu.VMEM((KG, HID), jnp.float32) for _ in range(NB)],
        pltpu.VMEM((HROWS, HR), jnp.float32),
        pltpu.SemaphoreType.DMA,
        [pltpu.SemaphoreType.DMA for _ in range(NB)],
        [pltpu.SemaphoreType.DMA for _ in range(NB)],
    ],
    compiler_params=_SC_PARAMS,
)
def _sc_gather(*refs):
    _sc_gather_body(*refs)


def _sc_scatter_body(m_hbm, dst_hbm, z_hbm, sums_hbm, idxg0, idxg1, rows,
                     acc_sh, sem_i0, sem_i1, sems_l, sems_s):
    c = lax.axis_index("c")
    s = lax.axis_index("s")
    wid = c * NS + s
    base = wid * EPW
    idxg = [idxg0, idxg1]
    sem_ig = [sem_i0, sem_i1]

    pltpu.async_copy(dst_hbm.at[wid, 0], idxg[0], sem_ig[0])

    pltpu.sync_copy(z_hbm, rows[0])
    for k in range(RPT // KS):
        r0 = s * RPT + k * KS
        pltpu.sync_copy(rows[0], acc_sh.at[pl.ds(r0, KS)])
    plsc.subcore_barrier()

    def group(g, carry):
        p = g % 2
        for b in range(NB):
            i = g * NB + b

            @pl.when(g > 0)
            def _():
                pltpu.make_async_copy(
                    rows[b], acc_sh.at[idxg0.at[b]], sems_s[b]
                ).wait()

            pltpu.async_copy(m_hbm.at[pl.ds(base + i * KS, KS)], rows[b],
                             sems_l[b])
        @pl.when(g + 1 < NGRP_S)
        def _():
            @pl.when(p == 0)
            def _():
                pltpu.async_copy(dst_hbm.at[wid, g + 1], idxg[1], sem_ig[1])

            @pl.when(p == 1)
            def _():
                pltpu.async_copy(dst_hbm.at[wid, g + 1], idxg[0], sem_ig[0])

        @pl.when(p == 0)
        def _():
            pltpu.make_async_copy(dst_hbm.at[wid, g], idxg[0], sem_ig[0]).wait()

        @pl.when(p == 1)
        def _():
            pltpu.make_async_copy(dst_hbm.at[wid, g], idxg[1], sem_ig[1]).wait()

        for b in range(NB):
            pltpu.make_async_copy(
                m_hbm.at[pl.ds(base, KS)], rows[b], sems_l[b]).wait()

            @pl.when(p == 0)
            def _():
                pltpu.async_copy(rows[b], acc_sh.at[idxg0.at[b]], sems_s[b],
                                 add=True)

            @pl.when(p == 1)
            def _():
                pltpu.async_copy(rows[b], acc_sh.at[idxg1.at[b]], sems_s[b],
                                 add=True)
        return carry

    lax.fori_loop(0, NGRP_S, group, 0)

    for b in range(NB):
        pltpu.make_async_copy(rows[b], acc_sh.at[idxg0.at[b]], sems_s[b]).wait()
    plsc.subcore_barrier()

    for k in range(RPT // KS):
        r0 = s * RPT + k * KS
        pltpu.sync_copy(acc_sh.at[pl.ds(r0, KS)], rows[0])
        pltpu.sync_copy(rows[0], sums_hbm.at[c, pl.ds(r0, KS)])


@functools.partial(
    pl.kernel,
    out_type=jax.ShapeDtypeStruct((NC, NP, HID), jnp.float32),
    mesh=_SC_MESH,
    scratch_types=[
        pltpu.VMEM((NB, KS), jnp.int32),
        pltpu.VMEM((NB, KS), jnp.int32),
        [pltpu.VMEM((KS, HID), jnp.float32) for _ in range(NB)],
        pltpu.VMEM_SHARED((NP, HID), jnp.float32),
        pltpu.SemaphoreType.DMA,
        pltpu.SemaphoreType.DMA,
        [pltpu.SemaphoreType.DMA for _ in range(NB)],
        [pltpu.SemaphoreType.DMA for _ in range(NB)],
    ],
    compiler_params=_SC_PARAMS,
)
def _sc_scatter(*refs):
    _sc_scatter_body(*refs)


def _xp_body(x_ref, w_ref, b_ref, o_ref):
    o_ref[...] = (
        jnp.dot(x_ref[...], w_ref[...], preferred_element_type=jnp.float32)
        + b_ref[...]
    )


def _node_precompute(x, W1a, b1):
    grid = (10,)
    return pl.pallas_call(
        _xp_body,
        grid=grid,
        in_specs=[
            pl.BlockSpec((N // 10, D_IN), lambda i: (i, 0)),
            pl.BlockSpec((D_IN, HID), lambda i: (0, 0)),
            pl.BlockSpec((1, HID), lambda i: (0, 0)),
        ],
        out_specs=pl.BlockSpec((N // 10, HID), lambda i: (i, 0)),
        out_shape=jax.ShapeDtypeStruct((N, HID), jnp.float32),
    )(x, W1a, b1)


def _mlp_body(e1_ref, ea_ref, w1b_ref, w2_ref, b2_ref, w3_ref, b3_ref,
              w4_ref, b4_ref, m_ref):
    h1 = jnp.maximum(
        e1_ref[...]
        + jnp.dot(ea_ref[...], w1b_ref[...], preferred_element_type=jnp.float32),
        0.0,
    )
    h2 = jnp.maximum(
        jnp.dot(h1, w2_ref[...], preferred_element_type=jnp.float32) + b2_ref[...],
        0.0,
    )
    h3 = jnp.maximum(
        jnp.dot(h2, w3_ref[...], preferred_element_type=jnp.float32) + b3_ref[...],
        0.0,
    )
    m_ref[...] = jnp.maximum(
        jnp.dot(h3, w4_ref[...], preferred_element_type=jnp.float32) + b4_ref[...],
        0.0,
    )


def _edge_mlp(e1, edge_attr, W1b, W2, b2, W3, b3, W4, b4):
    grid = (E // MLP_BLOCK,)
    return pl.pallas_call(
        _mlp_body,
        grid=grid,
        in_specs=[
            pl.BlockSpec((MLP_BLOCK, HID), lambda i: (i, 0)),
            pl.BlockSpec((MLP_BLOCK, D_EDGE), lambda i: (i, 0)),
            pl.BlockSpec((D_EDGE, HID), lambda i: (0, 0)),
            pl.BlockSpec((HID, 2 * HID), lambda i: (0, 0)),
            pl.BlockSpec((1, 2 * HID), lambda i: (0, 0)),
            pl.BlockSpec((2 * HID, HID), lambda i: (0, 0)),
            pl.BlockSpec((1, HID), lambda i: (0, 0)),
            pl.BlockSpec((HID, D_OUT), lambda i: (0, 0)),
            pl.BlockSpec((1, D_OUT), lambda i: (0, 0)),
        ],
        out_specs=pl.BlockSpec((MLP_BLOCK, D_OUT), lambda i: (i, 0)),
        out_shape=jax.ShapeDtypeStruct((E, D_OUT), jnp.float32),
    )(e1, edge_attr, W1b, W2, b2, W3, b3, W4, b4)


def _combine_body(s_ref, c_ref, o_ref):
    total = s_ref[0] + s_ref[1]
    cnt = jnp.sum(c_ref[...], axis=0)
    denom = jnp.maximum(cnt, 1.0)
    o_ref[...] = total / denom[:, None]


def _combine(sums, cnt):
    grid = (10,)
    blk = NP // 10
    out = pl.pallas_call(
        _combine_body,
        grid=grid,
        in_specs=[
            pl.BlockSpec((2, blk, D_OUT), lambda i: (0, i, 0)),
            pl.BlockSpec((32, blk), lambda i: (0, i)),
        ],
        out_specs=pl.BlockSpec((blk, D_OUT), lambda i: (i, 0)),
        out_shape=jax.ShapeDtypeStruct((NP, D_OUT), jnp.float32),
    )(sums, cnt)
    return out[:N]


def kernel(x, edge_index, edge_attr, W1, b1, W2, b2, W3, b3, W4, b4):
    dst = edge_index[1].astype(jnp.int32)
    W1a = W1[:D_IN]
    W1b = W1[D_IN:]
    b1r = b1.reshape(1, HID)
    b2r = b2.reshape(1, 2 * HID)
    b3r = b3.reshape(1, HID)
    b4r = b4.reshape(1, D_OUT)

    xp = _node_precompute(x, W1a, b1r)

    dst3 = dst.reshape(NW, NITER, KG)
    e1, cnt3 = _sc_gather(xp, dst3)
    m = _edge_mlp(e1, edge_attr, W1b, W2, b2r, W3, b3r, W4, b4r)
    zeros = jnp.zeros((KS, HID), jnp.float32)
    dst4s = dst.reshape(NW, NGRP_S, NB, KS)
    sums = _sc_scatter(m, dst4s, zeros)
    cnt = cnt3.reshape(NW, NP)
    return _combine(sums, cnt)


# speedup vs baseline: 4.4420x; 1.4448x over previous
import functools

import jax
import jax.numpy as jnp
from jax import lax
from jax.experimental import pallas as pl
from jax.experimental.pallas import tpu as pltpu
from jax.experimental.pallas import tpu_sc as plsc

N = 10000
E = 320000
D_IN = 128
D_EDGE = 16
HID = 128
D_OUT = 128

MLP_BLOCK = 1280

NC = 2
NS = 16
NW = NC * NS
EPW = E // NW
KG = 80
NITER = EPW // KG
NB = 5
NGRP = NITER // NB
NP = 10240
NB_S = 3
NGRP_S = NITER // NB_S
RPT = NP // NS
HROWS = 5
HR = 2048

_SC_MESH = plsc.VectorSubcoreMesh(
    core_axis_name="c", subcore_axis_name="s", num_cores=NC, num_subcores=NS
)
_SC_PARAMS = pltpu.CompilerParams(needs_layout_passes=False)


def _sc_gather_body(xp_hbm, dst_hbm, e1_hbm, cnt_hbm, idx_v, rows, hist_v,
                    sem_i, sems_g, sems_w):
    c = lax.axis_index("c")
    s = lax.axis_index("s")
    wid = c * NS + s
    base = wid * EPW

    idx_load = pltpu.async_copy(dst_hbm.at[wid], idx_v, sem_i)

    zeros16 = jnp.zeros((16,), jnp.float32)

    def zero_hist(i, carry):
        for r in range(HROWS):
            hist_v[r, pl.ds(i * 16, 16)] = zeros16
        return carry

    lax.fori_loop(0, HR // 16, zero_hist, 0)
    idx_load.wait()

    ones = jnp.ones((16,), jnp.float32)

    def hist_update(i):
        for j in range(KG // 16):
            idxs = idx_v[i, pl.ds(j * 16, 16)]
            plsc.addupdate_scatter(
                hist_v, [lax.shift_right_logical(idxs, 11),
                         lax.bitwise_and(idxs, HR - 1)], ones)

    def group(g, carry):
        descs = []
        for b in range(NB):
            i = g * NB + b

            @pl.when(g > 0)
            def _():
                pltpu.make_async_copy(
                    rows[b], e1_hbm.at[pl.ds(base + i * KG, KG)], sems_w[b]
                ).wait()

            descs.append(
                pltpu.async_copy(xp_hbm.at[idx_v.at[i]], rows[b], sems_g[b])
            )
        for b in range(NB):
            i = g * NB + b
            hist_update(i)
            descs[b].wait()
            pltpu.async_copy(rows[b], e1_hbm.at[pl.ds(base + i * KG, KG)],
                             sems_w[b])
        return carry

    lax.fori_loop(0, NGRP, group, 0)

    for b in range(NB):
        pltpu.make_async_copy(
            rows[b], e1_hbm.at[pl.ds(base, KG)], sems_w[b]
        ).wait()
    pltpu.sync_copy(hist_v, cnt_hbm.at[wid])


@functools.partial(
    pl.kernel,
    out_type=(
        jax.ShapeDtypeStruct((E, HID), jnp.float32),
        jax.ShapeDtypeStruct((NW, HROWS, HR), jnp.float32),
    ),
    mesh=_SC_MESH,
    scratch_types=[
        pltpu.VMEM((NITER, KG), jnp.int32),
        [pltpu.VMEM((KG, HID), jnp.float32) for _ in range(NB)],
        pltpu.VMEM((HROWS, HR), jnp.float32),
        pltpu.SemaphoreType.DMA,
        [pltpu.SemaphoreType.DMA for _ in range(NB)],
        [pltpu.SemaphoreType.DMA for _ in range(NB)],
    ],
    compiler_params=_SC_PARAMS,
)
def _sc_gather(*refs):
    _sc_gather_body(*refs)


def _sc_scatter_body(m_hbm, dst_hbm, z_hbm, sums_hbm, idx_v, rows,
                     acc_sh, sem_i, sems_l, sems_s):
    c = lax.axis_index("c")
    s = lax.axis_index("s")
    wid = c * NS + s
    base = wid * EPW

    idx_load = pltpu.async_copy(dst_hbm.at[wid], idx_v, sem_i)

    pltpu.sync_copy(z_hbm, rows[0])
    for k in range(RPT // KG):
        r0 = s * RPT + k * KG
        pltpu.sync_copy(rows[0], acc_sh.at[pl.ds(r0, KG)])
    idx_load.wait()
    plsc.subcore_barrier()

    def group(g, carry):
        for b in range(NB_S):
            i = g * NB_S + b

            @pl.when(g > 0)
            def _():
                pltpu.make_async_copy(
                    rows[b], acc_sh.at[idx_v.at[i]], sems_s[b]
                ).wait()

            pltpu.async_copy(m_hbm.at[pl.ds(base + i * KG, KG)], rows[b],
                             sems_l[b])
        for b in range(NB_S):
            i = g * NB_S + b
            pltpu.make_async_copy(
                m_hbm.at[pl.ds(base, KG)], rows[b], sems_l[b]).wait()
            pltpu.async_copy(rows[b], acc_sh.at[idx_v.at[i]], sems_s[b],
                             add=True)
        return carry

    lax.fori_loop(0, NGRP_S, group, 0)

    for b in range(NB_S):
        pltpu.make_async_copy(rows[b], acc_sh.at[idx_v.at[0]], sems_s[b]).wait()

    for i in (NGRP_S * NB_S, NGRP_S * NB_S + 1):
        pltpu.sync_copy(m_hbm.at[pl.ds(base + i * KG, KG)], rows[0])
        pltpu.sync_copy(rows[0], acc_sh.at[idx_v.at[i]], add=True)
    plsc.subcore_barrier()

    for k in range(RPT // KG):
        r0 = s * RPT + k * KG
        pltpu.sync_copy(acc_sh.at[pl.ds(r0, KG)], rows[0])
        pltpu.sync_copy(rows[0], sums_hbm.at[c, pl.ds(r0, KG)])


@functools.partial(
    pl.kernel,
    out_type=jax.ShapeDtypeStruct((NC, NP, HID), jnp.float32),
    mesh=_SC_MESH,
    scratch_types=[
        pltpu.VMEM((NITER, KG), jnp.int32),
        [pltpu.VMEM((KG, HID), jnp.float32) for _ in range(NB_S)],
        pltpu.VMEM_SHARED((NP, HID), jnp.float32),
        pltpu.SemaphoreType.DMA,
        [pltpu.SemaphoreType.DMA for _ in range(NB_S)],
        [pltpu.SemaphoreType.DMA for _ in range(NB_S)],
    ],
    compiler_params=_SC_PARAMS,
)
def _sc_scatter(*refs):
    _sc_scatter_body(*refs)


def _xp_body(x_ref, w_ref, b_ref, o_ref):
    o_ref[...] = (
        jnp.dot(x_ref[...], w_ref[...], preferred_element_type=jnp.float32)
        + b_ref[...]
    )


def _node_precompute(x, W1a, b1):
    grid = (10,)
    return pl.pallas_call(
        _xp_body,
        grid=grid,
        in_specs=[
            pl.BlockSpec((N // 10, D_IN), lambda i: (i, 0)),
            pl.BlockSpec((D_IN, HID), lambda i: (0, 0)),
            pl.BlockSpec((1, HID), lambda i: (0, 0)),
        ],
        out_specs=pl.BlockSpec((N // 10, HID), lambda i: (i, 0)),
        out_shape=jax.ShapeDtypeStruct((N, HID), jnp.float32),
    )(x, W1a, b1)


def _mlp_body(e1_ref, ea_ref, w1b_ref, w2_ref, b2_ref, w3_ref, b3_ref,
              w4_ref, b4_ref, m_ref):
    h1 = jnp.maximum(
        e1_ref[...]
        + jnp.dot(ea_ref[...], w1b_ref[...],
                  preferred_element_type=jnp.float32),
        0.0,
    ).astype(jnp.bfloat16)
    h2 = jnp.maximum(
        jnp.dot(h1, w2_ref[...], preferred_element_type=jnp.float32)
        + b2_ref[...],
        0.0,
    ).astype(jnp.bfloat16)
    h3 = jnp.maximum(
        jnp.dot(h2, w3_ref[...], preferred_element_type=jnp.float32)
        + b3_ref[...],
        0.0,
    ).astype(jnp.bfloat16)
    m_ref[...] = jnp.maximum(
        jnp.dot(h3, w4_ref[...], preferred_element_type=jnp.float32) + b4_ref[...],
        0.0,
    )


def _edge_mlp(e1, edge_attr, W1b, W2, b2, W3, b3, W4, b4):
    grid = (E // MLP_BLOCK,)
    return pl.pallas_call(
        _mlp_body,
        grid=grid,
        in_specs=[
            pl.BlockSpec((MLP_BLOCK, HID), lambda i: (i, 0)),
            pl.BlockSpec((MLP_BLOCK, D_EDGE), lambda i: (i, 0)),
            pl.BlockSpec((D_EDGE, HID), lambda i: (0, 0)),
            pl.BlockSpec((HID, 2 * HID), lambda i: (0, 0)),
            pl.BlockSpec((1, 2 * HID), lambda i: (0, 0)),
            pl.BlockSpec((2 * HID, HID), lambda i: (0, 0)),
            pl.BlockSpec((1, HID), lambda i: (0, 0)),
            pl.BlockSpec((HID, D_OUT), lambda i: (0, 0)),
            pl.BlockSpec((1, D_OUT), lambda i: (0, 0)),
        ],
        out_specs=pl.BlockSpec((MLP_BLOCK, D_OUT), lambda i: (i, 0)),
        out_shape=jax.ShapeDtypeStruct((E, D_OUT), jnp.float32),
    )(e1, edge_attr, W1b, W2, b2, W3, b3, W4, b4)


def _combine_body(s_ref, c_ref, o_ref):
    total = s_ref[0] + s_ref[1]
    cnt = jnp.sum(c_ref[...], axis=0)
    denom = jnp.maximum(cnt, 1.0)
    o_ref[...] = total / denom[:, None]


def _combine(sums, cnt):
    grid = (10,)
    blk = NP // 10
    out = pl.pallas_call(
        _combine_body,
        grid=grid,
        in_specs=[
            pl.BlockSpec((2, blk, D_OUT), lambda i: (0, i, 0)),
            pl.BlockSpec((32, blk), lambda i: (0, i)),
        ],
        out_specs=pl.BlockSpec((blk, D_OUT), lambda i: (i, 0)),
        out_shape=jax.ShapeDtypeStruct((NP, D_OUT), jnp.float32),
    )(sums, cnt)
    return out[:N]


def kernel(x, edge_index, edge_attr, W1, b1, W2, b2, W3, b3, W4, b4):
    dst = edge_index[1].astype(jnp.int32)
    W1a = W1[:D_IN]
    W1b = W1[D_IN:]
    b1r = b1.reshape(1, HID)
    b2r = b2.reshape(1, 2 * HID)
    b3r = b3.reshape(1, HID)
    b4r = b4.reshape(1, D_OUT)

    xp = _node_precompute(x, W1a, b1r)

    dst3 = dst.reshape(NW, NITER, KG)
    e1, cnt3 = _sc_gather(xp, dst3)
    m = _edge_mlp(e1, edge_attr.astype(jnp.bfloat16), W1b.astype(jnp.bfloat16),
                  W2.astype(jnp.bfloat16), b2r,
                  W3.astype(jnp.bfloat16), b3r,
                  W4.astype(jnp.bfloat16), b4r)
    zeros = jnp.zeros((KG, HID), jnp.float32)
    sums = _sc_scatter(m, dst3, zeros)
    cnt = cnt3.reshape(NW, NP)
    return _combine(sums, cnt)
